# baseline (speedup 1.0000x reference)
import functools

import jax
import jax.numpy as jnp
from jax import lax
from jax.experimental import pallas as pl
from jax.experimental.pallas import tpu as pltpu
from jax.experimental.pallas import tpu_sc as plsc

_VOCAB = 100000
_EMBED = 300
_EMBED_PAD = 384
_LENGTH = 200
_BATCH = 4096
_TOTAL = _LENGTH * _BATCH

_NSLAB = 4
_SLAB_L = _LENGTH // _NSLAB
_SLAB = _SLAB_L * _BATCH

_NC = 2
_NS = 16
_NW = _NC * _NS
_PER_W = _SLAB // _NW
_CHUNK = 128
_NCHUNK = _PER_W // _CHUNK

_STRIP = _EMBED - 256
_SCOLS = (0, 16, _STRIP - 16)

_ADD_ROWS = 1000


def _add_body(v_ref, p_ref, o_ref):
    o_ref[:, : _EMBED] = v_ref[...] + p_ref[...]
    o_ref[:, _EMBED :] = jnp.zeros(
        (_ADD_ROWS, _EMBED_PAD - _EMBED), jnp.float32
    )


def _combine_tables(vector_table, position_table):
    in_spec = pl.BlockSpec((_ADD_ROWS, _EMBED), lambda i: (i, 0))
    out_spec = pl.BlockSpec((_ADD_ROWS, _EMBED_PAD), lambda i: (i, 0))
    return pl.pallas_call(
        _add_body,
        grid=(_VOCAB // _ADD_ROWS,),
        in_specs=[in_spec, in_spec],
        out_specs=out_spec,
        out_shape=jax.ShapeDtypeStruct((_VOCAB, _EMBED_PAD), jnp.float32),
    )(vector_table, position_table)


def _gather_body(tbl_hbm, idx_hbm, out_hbm, idx_v, buf_v, strip_v, gsem, wsem, ssem):
    wid = lax.axis_index("s") * _NC + lax.axis_index("c")
    base = wid * _PER_W
    pltpu.sync_copy(idx_hbm.at[wid], idx_v)

    def start_gather(j, slot):
        pltpu.async_copy(tbl_hbm.at[idx_v.at[j]], buf_v.at[slot], gsem)

    def rows_ref(j):
        r0 = base + j * _CHUNK
        return out_hbm.at[r0 // _BATCH, pl.ds(lax.rem(r0, _BATCH), _CHUNK)]

    def wait_gather(j, slot):
        pltpu.make_async_copy(tbl_hbm.at[idx_v.at[j]], buf_v.at[slot], gsem).wait()

    def start_write(j, slot):
        pltpu.async_copy(
            buf_v.at[slot].at[:, pl.ds(0, 256)],
            rows_ref(j).at[:, pl.ds(0, 256)],
            wsem,
        )

    def wait_write(j, slot):
        pltpu.make_async_copy(
            buf_v.at[slot].at[:, pl.ds(0, 256)],
            rows_ref(j).at[:, pl.ds(0, 256)],
            wsem,
        ).wait()

    def start_strip(j):
        pltpu.async_copy(strip_v, rows_ref(j).at[:, pl.ds(256, _STRIP)], ssem)

    def wait_strip(j):
        pltpu.make_async_copy(
            strip_v, rows_ref(j).at[:, pl.ds(256, _STRIP)], ssem
        ).wait()

    def unpack_strip(slot):
        def row(r, carry):
            for col in _SCOLS:
                strip_v[r, pl.ds(col, 16)] = buf_v[slot, r, pl.ds(256 + col, 16)]
            return carry

        lax.fori_loop(0, _CHUNK, row, 0)

    start_gather(0, 0)

    def body(j, carry):
        slot = lax.rem(j, 2)
        wait_gather(j, slot)
        nj = j + 1

        @pl.when(j >= 1)
        def _():
            wait_strip(j - 1)

        @pl.when(nj < _NCHUNK)
        def _():
            @pl.when(j >= 1)
            def _():
                wait_write(j - 1, 1 - slot)

            start_gather(nj, 1 - slot)

        start_write(j, slot)
        unpack_strip(slot)
        start_strip(j)
        return carry

    lax.fori_loop(0, _NCHUNK, body, 0)

    last = _NCHUNK - 1
    wait_write(last - 1, lax.rem(last - 1, 2))
    wait_write(last, lax.rem(last, 2))
    wait_strip(last)


def _make_gather():
    return functools.partial(
        pl.kernel,
        out_type=jax.ShapeDtypeStruct((_SLAB_L, _BATCH, _EMBED), jnp.float32),
        mesh=plsc.VectorSubcoreMesh(core_axis_name="c", subcore_axis_name="s"),
        scratch_types=[
            pltpu.VMEM((_NCHUNK, _CHUNK), jnp.int32),
            pltpu.VMEM((2, _CHUNK, _EMBED_PAD), jnp.float32),
            pltpu.VMEM((_CHUNK, _STRIP), jnp.float32),
            pltpu.SemaphoreType.DMA,
            pltpu.SemaphoreType.DMA,
            pltpu.SemaphoreType.DMA,
        ],
    )(_gather_body)


def kernel(x, vector_table, position_table):
    sum_table = _combine_tables(vector_table, position_table)
    xf = x.reshape(-1).astype(jnp.int32).reshape(_NSLAB, _NW, _NCHUNK, _CHUNK)
    gather = _make_gather()
    slabs = [gather(sum_table, xf[k]) for k in range(_NSLAB)]
    return jnp.concatenate(slabs, axis=0)


# speedup vs baseline: 1.3983x; 1.3983x over previous
import functools

import jax
import jax.numpy as jnp
from jax import lax
from jax.experimental import pallas as pl
from jax.experimental.pallas import tpu as pltpu
from jax.experimental.pallas import tpu_sc as plsc

_VOCAB = 100000
_EMBED = 300
_EMBED_PAD = 384
_LENGTH = 200
_BATCH = 4096
_TOTAL = _LENGTH * _BATCH

_NC = 2
_NS = 16
_NW = _NC * _NS
_PER_W = _TOTAL // _NW
_CHUNK = 128
_NCHUNK = _PER_W // _CHUNK
_IDXROWS = 40

_STRIP = _EMBED - 256
_SCOLS = (0, 16, _STRIP - 16)

_ADD_ROWS = 1000


def _add_body(v_ref, p_ref, o_ref):
    o_ref[:, : _EMBED] = v_ref[...] + p_ref[...]
    o_ref[:, _EMBED :] = jnp.zeros(
        (_ADD_ROWS, _EMBED_PAD - _EMBED), jnp.float32
    )


def _combine_tables(vector_table, position_table):
    in_spec = pl.BlockSpec((_ADD_ROWS, _EMBED), lambda i: (i, 0))
    out_spec = pl.BlockSpec((_ADD_ROWS, _EMBED_PAD), lambda i: (i, 0))
    return pl.pallas_call(
        _add_body,
        grid=(_VOCAB // _ADD_ROWS,),
        in_specs=[in_spec, in_spec],
        out_specs=out_spec,
        out_shape=jax.ShapeDtypeStruct((_VOCAB, _EMBED_PAD), jnp.float32),
    )(vector_table, position_table)


def _gather_body(tbl_hbm, idx_hbm, out_hbm, idx_v, buf_v, strip_v, gsem, wsem, ssem):
    wid = lax.axis_index("s") * _NC + lax.axis_index("c")
    base = wid * _PER_W

    def stage(p):
        pltpu.sync_copy(idx_hbm.at[wid, pl.ds(p * _IDXROWS, _IDXROWS)], idx_v)

    def start_gather(j, slot):
        jl = lax.rem(j, _IDXROWS)
        pltpu.async_copy(tbl_hbm.at[idx_v.at[jl]], buf_v.at[slot], gsem)

    def rows_ref(j):
        return out_hbm.at[pl.ds(base + j * _CHUNK, _CHUNK)]

    def wait_gather(j, slot):
        jl = lax.rem(j, _IDXROWS)
        pltpu.make_async_copy(tbl_hbm.at[idx_v.at[jl]], buf_v.at[slot], gsem).wait()

    def start_write(j, slot):
        pltpu.async_copy(
            buf_v.at[slot].at[:, pl.ds(0, 256)],
            rows_ref(j).at[:, pl.ds(0, 256)],
            wsem,
        )

    def wait_write(j, slot):
        pltpu.make_async_copy(
            buf_v.at[slot].at[:, pl.ds(0, 256)],
            rows_ref(j).at[:, pl.ds(0, 256)],
            wsem,
        ).wait()

    def start_strip(j):
        pltpu.async_copy(strip_v, rows_ref(j).at[:, pl.ds(256, _STRIP)], ssem)

    def wait_strip(j):
        pltpu.make_async_copy(
            strip_v, rows_ref(j).at[:, pl.ds(256, _STRIP)], ssem
        ).wait()

    def unpack_strip(slot):
        def row(r, carry):
            for col in _SCOLS:
                strip_v[r, pl.ds(col, 16)] = buf_v[slot, r, pl.ds(256 + col, 16)]
            return carry

        lax.fori_loop(0, _CHUNK, row, 0)

    stage(0)
    start_gather(0, 0)

    def body(j, carry):
        slot = lax.rem(j, 2)
        wait_gather(j, slot)
        nj = j + 1

        @pl.when(jnp.logical_and(nj < _NCHUNK, lax.rem(nj, _IDXROWS) == 0))
        def _():
            stage(nj // _IDXROWS)

        @pl.when(j >= 1)
        def _():
            wait_strip(j - 1)

        @pl.when(nj < _NCHUNK)
        def _():
            @pl.when(j >= 1)
            def _():
                wait_write(j - 1, 1 - slot)

            start_gather(nj, 1 - slot)

        start_write(j, slot)
        unpack_strip(slot)
        start_strip(j)
        return carry

    lax.fori_loop(0, _NCHUNK, body, 0)

    last = _NCHUNK - 1
    wait_write(last - 1, lax.rem(last - 1, 2))
    wait_write(last, lax.rem(last, 2))
    wait_strip(last)


def _make_gather():
    return functools.partial(
        pl.kernel,
        out_type=jax.ShapeDtypeStruct((_TOTAL, _EMBED), jnp.float32),
        mesh=plsc.VectorSubcoreMesh(core_axis_name="c", subcore_axis_name="s"),
        scratch_types=[
            pltpu.VMEM((_IDXROWS, _CHUNK), jnp.int32),
            pltpu.VMEM((2, _CHUNK, _EMBED_PAD), jnp.float32),
            pltpu.VMEM((_CHUNK, _STRIP), jnp.float32),
            pltpu.SemaphoreType.DMA,
            pltpu.SemaphoreType.DMA,
            pltpu.SemaphoreType.DMA,
        ],
    )(_gather_body)


def kernel(x, vector_table, position_table):
    sum_table = _combine_tables(vector_table, position_table)
    xf = x.reshape(-1).astype(jnp.int32).reshape(_NW, _NCHUNK, _CHUNK)
    out = _make_gather()(sum_table, xf)
    return out.reshape(_LENGTH, _BATCH, _EMBED)


# speedup vs baseline: 1.3999x; 1.0012x over previous
import functools

import jax
import jax.numpy as jnp
from jax import lax
from jax.experimental import pallas as pl
from jax.experimental.pallas import tpu as pltpu
from jax.experimental.pallas import tpu_sc as plsc

_VOCAB = 100000
_EMBED = 300
_EMBED_PAD = 384
_LENGTH = 200
_BATCH = 4096
_TOTAL = _LENGTH * _BATCH

_NC = 2
_NS = 16
_NW = _NC * _NS
_PER_W = _TOTAL // _NW
_CHUNK = 128
_NCHUNK = _PER_W // _CHUNK
_IDXROWS = 40

_STRIP = _EMBED - 256
_SCOLS = (0, 16, _STRIP - 16)

_ADD_ROWS = 1000


def _add_body(v_ref, p_ref, o_ref):
    o_ref[:, : _EMBED] = v_ref[...] + p_ref[...]
    o_ref[:, _EMBED :] = jnp.zeros(
        (_ADD_ROWS, _EMBED_PAD - _EMBED), jnp.float32
    )


def _combine_tables(vector_table, position_table):
    in_spec = pl.BlockSpec((_ADD_ROWS, _EMBED), lambda i: (i, 0))
    out_spec = pl.BlockSpec((_ADD_ROWS, _EMBED_PAD), lambda i: (i, 0))
    return pl.pallas_call(
        _add_body,
        grid=(_VOCAB // _ADD_ROWS,),
        in_specs=[in_spec, in_spec],
        out_specs=out_spec,
        out_shape=jax.ShapeDtypeStruct((_VOCAB, _EMBED_PAD), jnp.float32),
    )(vector_table, position_table)


def _gather_body(tbl_hbm, idx_hbm, out_hbm, idx_v, buf_v, strip_v, gsem, wsem, ssem):
    wid = lax.axis_index("s") * _NC + lax.axis_index("c")
    base = wid * _PER_W

    def stage(p):
        pltpu.sync_copy(idx_hbm.at[wid, pl.ds(p * _IDXROWS, _IDXROWS)], idx_v)

    def start_gather(j, slot):
        jl = lax.rem(j, _IDXROWS)
        pltpu.async_copy(tbl_hbm.at[idx_v.at[jl]], buf_v.at[slot], gsem)

    def rows_ref(j):
        return out_hbm.at[pl.ds(base + j * _CHUNK, _CHUNK)]

    def wait_gather(j, slot):
        jl = lax.rem(j, _IDXROWS)
        pltpu.make_async_copy(tbl_hbm.at[idx_v.at[jl]], buf_v.at[slot], gsem).wait()

    def start_write(j, slot):
        pltpu.async_copy(
            buf_v.at[slot].at[:, pl.ds(0, 256)],
            rows_ref(j).at[:, pl.ds(0, 256)],
            wsem,
        )

    def wait_write(j, slot):
        pltpu.make_async_copy(
            buf_v.at[slot].at[:, pl.ds(0, 256)],
            rows_ref(j).at[:, pl.ds(0, 256)],
            wsem,
        ).wait()

    def start_strip(j):
        pltpu.async_copy(strip_v, rows_ref(j).at[:, pl.ds(256, _STRIP)], ssem)

    def wait_strip(j):
        pltpu.make_async_copy(
            strip_v, rows_ref(j).at[:, pl.ds(256, _STRIP)], ssem
        ).wait()

    def unpack_strip(slot):
        def row(r, carry):
            for col in _SCOLS:
                strip_v[r, pl.ds(col, 16)] = buf_v[slot, r, pl.ds(256 + col, 16)]
            return carry

        lax.fori_loop(0, _CHUNK, row, 0)

    stage(0)
    start_gather(0, 0)

    def body(j, carry):
        slot = lax.rem(j, 2)
        nj = j + 1
        boundary = lax.rem(nj, _IDXROWS) == 0

        @pl.when(jnp.logical_and(nj < _NCHUNK, jnp.logical_not(boundary)))
        def _():
            @pl.when(j >= 1)
            def _():
                wait_write(j - 1, 1 - slot)

            start_gather(nj, 1 - slot)

        wait_gather(j, slot)

        @pl.when(jnp.logical_and(nj < _NCHUNK, boundary))
        def _():
            stage(nj // _IDXROWS)
            wait_write(j - 1, 1 - slot)
            start_gather(nj, 1 - slot)

        @pl.when(j >= 1)
        def _():
            wait_strip(j - 1)

        start_write(j, slot)
        unpack_strip(slot)
        start_strip(j)
        return carry

    lax.fori_loop(0, _NCHUNK, body, 0)

    last = _NCHUNK - 1
    wait_write(last - 1, lax.rem(last - 1, 2))
    wait_write(last, lax.rem(last, 2))
    wait_strip(last)


def _make_gather():
    return functools.partial(
        pl.kernel,
        out_type=jax.ShapeDtypeStruct((_TOTAL, _EMBED), jnp.float32),
        mesh=plsc.VectorSubcoreMesh(core_axis_name="c", subcore_axis_name="s"),
        scratch_types=[
            pltpu.VMEM((_IDXROWS, _CHUNK), jnp.int32),
            pltpu.VMEM((2, _CHUNK, _EMBED_PAD), jnp.float32),
            pltpu.VMEM((_CHUNK, _STRIP), jnp.float32),
            pltpu.SemaphoreType.DMA,
            pltpu.SemaphoreType.DMA,
            pltpu.SemaphoreType.DMA,
        ],
    )(_gather_body)


def kernel(x, vector_table, position_table):
    sum_table = _combine_tables(vector_table, position_table)
    xf = x.reshape(-1).astype(jnp.int32).reshape(_NW, _NCHUNK, _CHUNK)
    out = _make_gather()(sum_table, xf)
    return out.reshape(_LENGTH, _BATCH, _EMBED)
